# Initial kernel scaffold; baseline (speedup 1.0000x reference)
import jax
import jax.numpy as jnp
from jax.experimental import pallas as pl


def kernel(user_ids, item_ids, user_emb, item_emb, adj_row, adj_col, adj_vals):
    raise NotImplementedError("write your pallas kernel here")


# speedup vs baseline: 1.8307x; 1.8307x over previous
import functools

import jax
import jax.numpy as jnp
from jax import lax
from jax.experimental import pallas as pl
from jax.experimental.pallas import tpu as pltpu
from jax.experimental.pallas import tpu_sc as plsc

_U = 50000
_I = 50000
_N = _U + _I
_D = 64
_NNZ = 1250000
_B = 4096

_NC = 2
_NS = 16
_L = 16

_CH = 25600
_NPAD = 4 * _CH
_ZROWS = _CH // _NS

_EC = 1024
_SUB = 128
_ET = 77 * _EC
_NNZ_PAD = _NS * _ET


def _spmm_body(eprev, rows, cols, vals, zrows, enext,
               rbuf, cbuf, vbuf, ribuf, rowbuf, acc):
    cid = lax.axis_index("c")
    sid = lax.axis_index("s")
    ebase = sid * _ET

    for sweep in range(2):
        base = (sweep * _NC + cid) * _CH

        pltpu.sync_copy(zrows, acc.at[pl.ds(sid * _ZROWS, _ZROWS)])
        plsc.subcore_barrier()

        def chunk_body(c, carry):
            off = ebase + c * _EC
            pltpu.sync_copy(rows.at[pl.ds(off, _EC)], rbuf)
            pltpu.sync_copy(cols.at[pl.ds(off, _EC)], cbuf)
            pltpu.sync_copy(vals.at[pl.ds(off, _EC)], vbuf)

            def sub_body(u, carry2):
                pltpu.sync_copy(
                    eprev.at[cbuf.at[pl.ds(u * _SUB, _SUB)]], rowbuf)

                def grp_body(g, carry3):
                    e0 = u * _SUB + g * _L
                    r = rbuf[pl.ds(e0, _L)]
                    v = vbuf[pl.ds(e0, _L)]
                    rel = r - base
                    m = (rel >= 0) & (rel < _CH)
                    ridx = jnp.where(m, rel, 0)
                    vm = jnp.where(m, v, 0.0)
                    ribuf[0, pl.ds(g * _L, _L)] = ridx
                    for e in range(_L):
                        vs = lax.squeeze(lax.slice(vm, (e,), (e + 1,)), (0,))
                        vsp = jnp.broadcast_to(vs, (_L,))
                        eidx = g * _L + e
                        for d in range(_D // _L):
                            rowbuf[eidx, pl.ds(d * _L, _L)] = (
                                rowbuf[eidx, pl.ds(d * _L, _L)] * vsp)
                    return carry3

                lax.fori_loop(0, _SUB // _L, grp_body, 0)
                pltpu.sync_copy(rowbuf, acc.at[ribuf.at[0]], add=True)
                return carry2

            lax.fori_loop(0, _EC // _SUB, sub_body, 0)
            return carry

        lax.fori_loop(0, _ET // _EC, chunk_body, 0)
        plsc.subcore_barrier()

        pltpu.sync_copy(acc.at[pl.ds(sid * _ZROWS, _ZROWS)],
                        enext.at[pl.ds(base + sid * _ZROWS, _ZROWS)])
        plsc.subcore_barrier()


def _final_body(e0, e1, e2, e3, uid, iid, out,
                ubuf, ibuf, u0, u1, u2, u3, i0, i1, i2, i3, obuf):
    cid = lax.axis_index("c")
    sid = lax.axis_index("s")
    wid = sid * _NC + cid
    nb = _B // (_NC * _NS)
    pltpu.sync_copy(uid.at[pl.ds(wid * nb, nb)], ubuf)
    pltpu.sync_copy(iid.at[pl.ds(wid * nb, nb)], ibuf)

    def adj_body(g, carry):
        ibuf[pl.ds(g * _L, _L)] = ibuf[pl.ds(g * _L, _L)] + _U
        return carry

    lax.fori_loop(0, nb // _L, adj_body, 0)

    for tab, dst in ((e0, u0), (e1, u1), (e2, u2), (e3, u3)):
        pltpu.sync_copy(tab.at[ubuf], dst)
    for tab, dst in ((e0, i0), (e1, i1), (e2, i2), (e3, i3)):
        pltpu.sync_copy(tab.at[ibuf], dst)

    lanes = jax.lax.broadcasted_iota(jnp.int32, (_L,), 0)

    def dot_body(g, carry):
        res = jnp.zeros((_L,), jnp.float32)
        for bb in range(_L):
            b = g * _L + bb
            accv = jnp.zeros((_L,), jnp.float32)
            for d in range(_D // _L):
                sl = pl.ds(d * _L, _L)
                fu = u0[b, sl] + u1[b, sl] + u2[b, sl] + u3[b, sl]
                fi = i0[b, sl] + i1[b, sl] + i2[b, sl] + i3[b, sl]
                accv = accv + fu * fi
            s = jnp.sum(accv) * (1.0 / 16.0)
            res = jnp.where(lanes == bb, jnp.broadcast_to(s, (_L,)), res)
        obuf[pl.ds(g * _L, _L)] = res
        return carry

    lax.fori_loop(0, nb // _L, dot_body, 0)
    pltpu.sync_copy(obuf, out.at[pl.ds(wid * nb, nb)])


def _make_spmm():
    mesh = plsc.VectorSubcoreMesh(core_axis_name="c", subcore_axis_name="s")
    return pl.kernel(
        _spmm_body,
        out_type=jax.ShapeDtypeStruct((_NPAD, _D), jnp.float32),
        mesh=mesh,
        compiler_params=pltpu.CompilerParams(
            use_tc_tiling_on_sc=False, needs_layout_passes=False),
        scratch_types=[
            pltpu.VMEM((_EC,), jnp.int32),
            pltpu.VMEM((_EC,), jnp.int32),
            pltpu.VMEM((_EC,), jnp.float32),
            pltpu.VMEM((1, _SUB), jnp.int32),
            pltpu.VMEM((_SUB, _D), jnp.float32),
            pltpu.VMEM_SHARED((_CH, _D), jnp.float32),
        ],
    )


def _make_final():
    mesh = plsc.VectorSubcoreMesh(core_axis_name="c", subcore_axis_name="s")
    nb = _B // (_NC * _NS)
    return pl.kernel(
        _final_body,
        out_type=jax.ShapeDtypeStruct((_B,), jnp.float32),
        mesh=mesh,
        compiler_params=pltpu.CompilerParams(
            use_tc_tiling_on_sc=False, needs_layout_passes=False),
        scratch_types=[
            pltpu.VMEM((nb,), jnp.int32),
            pltpu.VMEM((nb,), jnp.int32),
            pltpu.VMEM((nb, _D), jnp.float32),
            pltpu.VMEM((nb, _D), jnp.float32),
            pltpu.VMEM((nb, _D), jnp.float32),
            pltpu.VMEM((nb, _D), jnp.float32),
            pltpu.VMEM((nb, _D), jnp.float32),
            pltpu.VMEM((nb, _D), jnp.float32),
            pltpu.VMEM((nb, _D), jnp.float32),
            pltpu.VMEM((nb, _D), jnp.float32),
            pltpu.VMEM((nb,), jnp.float32),
        ],
    )


def kernel(user_ids, item_ids, user_emb, item_emb, adj_row, adj_col, adj_vals):
    e0 = jnp.concatenate([user_emb, item_emb], axis=0)
    e0 = jnp.pad(e0, ((0, _NPAD - _N), (0, 0)))
    pad = _NNZ_PAD - _NNZ
    rows = jnp.pad(adj_row, (0, pad))
    cols = jnp.pad(adj_col, (0, pad))
    vals = jnp.pad(adj_vals, (0, pad))
    zrows = jnp.zeros((_ZROWS, _D), jnp.float32)

    spmm = _make_spmm()
    e1 = spmm(e0, rows, cols, vals, zrows)
    e2 = spmm(e1, rows, cols, vals, zrows)
    e3 = spmm(e2, rows, cols, vals, zrows)

    final = _make_final()
    return final(e0, e1, e2, e3, user_ids, item_ids)


# speedup vs baseline: 4.4188x; 2.4137x over previous
import jax
import jax.numpy as jnp
from jax import lax
from jax.experimental import pallas as pl
from jax.experimental.pallas import tpu as pltpu
from jax.experimental.pallas import tpu_sc as plsc

_U = 50000
_I = 50000
_N = _U + _I
_D = 64
_NNZ = 1250000
_B = 4096

_NC = 2
_NS = 16
_L = 16
_NW = _NC * _NS

_CH = 25600
_NPAD = 4 * _CH
_ZROWS = _CH // _NS

_EB = 4096
_E32 = 40960
_NNZ32 = _NW * _E32
_F = 2048
_STG = 3072
_RS = 20 * _F + _STG

_BLK = 128
_SUB = 128


def _scalar(x):
    return lax.squeeze(lax.slice(x, (0,), (1,)), (0,))


def _part_body(rows, cols, vals, prow, pcol, pval, meta,
               rbufe, cbufe, vbufe, stgr, stgc, stgv, mbuf):
    cid = lax.axis_index("c")
    sid = lax.axis_index("s")
    wid = sid * _NC + cid
    lanes = lax.broadcasted_iota(jnp.int32, (_L,), 0)

    def outer(ob, carry):
        off = wid * _E32 + ob * _EB
        pltpu.sync_copy(rows.at[pl.ds(off, _EB)], rbufe)
        pltpu.sync_copy(cols.at[pl.ds(off, _EB)], cbufe)
        pltpu.sync_copy(vals.at[pl.ds(off, _EB)], vbufe)

        def inner(i, cr):
            r = rbufe[pl.ds(i * _L, _L)]
            c = cbufe[pl.ds(i * _L, _L)]
            v = vbufe[pl.ds(i * _L, _L)]
            k = r // _CH
            rel = r - k * _CH
            cr = list(cr)
            for kk in range(4):
                m = k == kk
                cnt = cr[2 * kk]
                fl = cr[2 * kk + 1]
                plsc.store_compressed(stgr.at[kk, pl.ds(cnt, _L)], rel, mask=m)
                plsc.store_compressed(stgc.at[kk, pl.ds(cnt, _L)], c, mask=m)
                plsc.store_compressed(stgv.at[kk, pl.ds(cnt, _L)], v, mask=m)
                cnt2 = cnt + _scalar(plsc.all_reduce_population_count(m))

                def flush(cc, ff):
                    dst = (wid * 4 + kk) * _RS + ff * _F
                    pltpu.sync_copy(stgr.at[kk, pl.ds(0, _F)],
                                    prow.at[pl.ds(dst, _F)])
                    pltpu.sync_copy(stgc.at[kk, pl.ds(0, _F)],
                                    pcol.at[pl.ds(dst, _F)])
                    pltpu.sync_copy(stgv.at[kk, pl.ds(0, _F)],
                                    pval.at[pl.ds(dst, _F)])
                    tr = stgr[kk, pl.ds(_F, _L)]
                    tc = stgc[kk, pl.ds(_F, _L)]
                    tv = stgv[kk, pl.ds(_F, _L)]
                    stgr[kk, pl.ds(0, _L)] = tr
                    stgc[kk, pl.ds(0, _L)] = tc
                    stgv[kk, pl.ds(0, _L)] = tv
                    return cc - _F, ff + 1

                cnt3, fl3 = lax.cond(cnt2 >= _F, flush,
                                     lambda cc, ff: (cc, ff), cnt2, fl)
                cr[2 * kk] = cnt3
                cr[2 * kk + 1] = fl3
            return tuple(cr)

        return lax.fori_loop(0, _EB // _L, inner, carry)

    z = jnp.int32(0)
    carry = lax.fori_loop(0, _E32 // _EB, outer,
                          (z, z, z, z, z, z, z, z))

    mvec = jnp.zeros((_L,), jnp.int32)
    zeros_i = jnp.zeros((_L,), jnp.int32)
    zeros_f = jnp.zeros((_L,), jnp.float32)
    for kk in range(4):
        cnt = carry[2 * kk]
        fl = carry[2 * kk + 1]

        def zpad(zi, c3, kk=kk):
            stgr[kk, pl.ds(cnt + zi * _L, _L)] = zeros_i
            stgc[kk, pl.ds(cnt + zi * _L, _L)] = zeros_i
            stgv[kk, pl.ds(cnt + zi * _L, _L)] = zeros_f
            return c3

        lax.fori_loop(0, 256 // _L, zpad, 0)
        dst = (wid * 4 + kk) * _RS + fl * _F
        pltpu.sync_copy(stgr.at[kk, pl.ds(0, _STG)], prow.at[pl.ds(dst, _STG)])
        pltpu.sync_copy(stgc.at[kk, pl.ds(0, _STG)], pcol.at[pl.ds(dst, _STG)])
        pltpu.sync_copy(stgv.at[kk, pl.ds(0, _STG)], pval.at[pl.ds(dst, _STG)])
        nblk2 = (fl * _F + cnt + 255) // 256
        mvec = jnp.where(lanes == kk, jnp.broadcast_to(nblk2, (_L,)), mvec)
    mbuf[pl.ds(0, _L)] = mvec
    pltpu.sync_copy(mbuf, meta.at[wid])


def _spmm_body(eprev, prow, pcol, pval, meta, zrows, enext,
               mbuf, cbufs, vbufs, rbufs, sbufs, rowbufs,
               semE, semG, semS):
    cid = lax.axis_index("c")
    sid = lax.axis_index("s")
    lanes = lax.broadcasted_iota(jnp.int32, (_L,), 0)
    acc = rowbufs[2]

    for sweep in range(2):
        c = sweep * _NC + cid
        base = c * _CH
        pltpu.sync_copy(zrows, acc.at[pl.ds(sid * _ZROWS, _ZROWS)])
        plsc.subcore_barrier()

        for li in range(2):
            src = sid * _NC + li
            pltpu.sync_copy(meta.at[src], mbuf)
            mv = mbuf[pl.ds(0, _L)]
            nblk2 = jnp.sum(jnp.where(lanes == c, mv, 0))
            nb = nblk2 * 2
            segbase = (src * 4 + c) * _RS

            def fire_edges(b, h):
                eoff = segbase + b * _BLK
                pltpu.async_copy(pcol.at[pl.ds(eoff, _BLK)], cbufs[h], semE[h])
                pltpu.async_copy(pval.at[pl.ds(eoff, _BLK)], vbufs[h], semE[h])
                for u in range(_BLK // _SUB):
                    pltpu.async_copy(
                        prow.at[pl.ds(eoff + u * _SUB, _SUB)],
                        rbufs[h].at[u], semE[h])

            def wait_edges(h):
                pltpu.make_async_copy(
                    pcol.at[pl.ds(0, _BLK)], cbufs[h], semE[h]).wait()
                pltpu.make_async_copy(
                    pval.at[pl.ds(0, _BLK)], vbufs[h], semE[h]).wait()
                for u in range(_BLK // _SUB):
                    pltpu.make_async_copy(
                        prow.at[pl.ds(0, _SUB)], rbufs[h].at[u],
                        semE[h]).wait()

            def drain_scatter(h):
                for u in range(_BLK // _SUB):
                    pltpu.make_async_copy(
                        eprev.at[pl.ds(0, _SUB)],
                        rowbufs[h].at[pl.ds(u * _SUB, _SUB)], semS[h]).wait()

            @pl.when(nblk2 > 0)
            def _():
                fire_edges(0, 0)
                fire_edges(1, 1)

            def blk_body(bb, carry):
                for h in range(2):
                    b = bb * 2 + h

                    @pl.when(bb > 0)
                    def _(h=h):
                        drain_scatter(h)

                    wait_edges(h)
                    descs = [
                        pltpu.async_copy(
                            eprev.at[cbufs[h].at[pl.ds(u * _SUB, _SUB)]],
                            rowbufs[h].at[pl.ds(u * _SUB, _SUB)], semG[h])
                        for u in range(_BLK // _SUB)]
                    for dsc in descs:
                        dsc.wait()

                    def idxcp(q, cr2, h=h):
                        for u in range(_BLK // _SUB):
                            sbufs[h][u, pl.ds(q * _L, _L)] = (
                                rbufs[h][u, pl.ds(q * _L, _L)])
                        return cr2

                    lax.fori_loop(0, _SUB // _L, idxcp, 0)

                    def scale(g, cr2, h=h):
                        e0 = g * _L
                        vv = vbufs[h][pl.ds(e0, _L)]
                        for e in range(_L):
                            vs = lax.squeeze(lax.slice(vv, (e,), (e + 1,)),
                                             (0,))
                            vsp = jnp.broadcast_to(vs, (_L,))
                            for d in range(_D // _L):
                                rowbufs[h][e0 + e, pl.ds(d * _L, _L)] = (
                                    rowbufs[h][e0 + e, pl.ds(d * _L, _L)]
                                    * vsp)
                        return cr2

                    lax.fori_loop(0, _BLK // _L, scale, 0)
                    for u in range(_BLK // _SUB):
                        pltpu.async_copy(
                            rowbufs[h].at[pl.ds(u * _SUB, _SUB)],
                            acc.at[sbufs[h].at[u]], semS[h], add=True)

                    @pl.when(b + 2 < nb)
                    def _(b=b, h=h):
                        fire_edges(b + 2, h)
                return carry

            lax.fori_loop(0, nblk2, blk_body, 0)

            @pl.when(nblk2 > 0)
            def _():
                drain_scatter(0)
                drain_scatter(1)

        plsc.subcore_barrier()
        pltpu.sync_copy(acc.at[pl.ds(sid * _ZROWS, _ZROWS)],
                        enext.at[pl.ds(base + sid * _ZROWS, _ZROWS)])
        plsc.subcore_barrier()


def _final_body(e0, e1, e2, e3, uid, iid, out,
                ubuf, ibuf, u0, u1, u2, u3, i0, i1, i2, i3, obuf):
    cid = lax.axis_index("c")
    sid = lax.axis_index("s")
    wid = sid * _NC + cid
    nb = _B // _NW
    pltpu.sync_copy(uid.at[pl.ds(wid * nb, nb)], ubuf)
    pltpu.sync_copy(iid.at[pl.ds(wid * nb, nb)], ibuf)

    def adj_body(g, carry):
        ibuf[pl.ds(g * _L, _L)] = ibuf[pl.ds(g * _L, _L)] + _U
        return carry

    lax.fori_loop(0, nb // _L, adj_body, 0)

    for tab, dst in ((e0, u0), (e1, u1), (e2, u2), (e3, u3)):
        pltpu.sync_copy(tab.at[ubuf], dst)
    for tab, dst in ((e0, i0), (e1, i1), (e2, i2), (e3, i3)):
        pltpu.sync_copy(tab.at[ibuf], dst)

    lanes = jax.lax.broadcasted_iota(jnp.int32, (_L,), 0)

    def dot_body(g, carry):
        res = jnp.zeros((_L,), jnp.float32)
        for bb in range(_L):
            b = g * _L + bb
            accv = jnp.zeros((_L,), jnp.float32)
            for d in range(_D // _L):
                sl = pl.ds(d * _L, _L)
                fu = u0[b, sl] + u1[b, sl] + u2[b, sl] + u3[b, sl]
                fi = i0[b, sl] + i1[b, sl] + i2[b, sl] + i3[b, sl]
                accv = accv + fu * fi
            s = jnp.sum(accv) * (1.0 / 16.0)
            res = jnp.where(lanes == bb, jnp.broadcast_to(s, (_L,)), res)
        obuf[pl.ds(g * _L, _L)] = res
        return carry

    lax.fori_loop(0, nb // _L, dot_body, 0)
    pltpu.sync_copy(obuf, out.at[pl.ds(wid * nb, nb)])


_MESH = plsc.VectorSubcoreMesh(core_axis_name="c", subcore_axis_name="s")
_PARAMS = pltpu.CompilerParams(
    use_tc_tiling_on_sc=False, needs_layout_passes=False)
_PTOT = _NW * 4 * _RS


def _make_part():
    return pl.kernel(
        _part_body,
        out_type=(
            jax.ShapeDtypeStruct((_PTOT,), jnp.int32),
            jax.ShapeDtypeStruct((_PTOT,), jnp.int32),
            jax.ShapeDtypeStruct((_PTOT,), jnp.float32),
            jax.ShapeDtypeStruct((_NW, _L), jnp.int32),
        ),
        mesh=_MESH,
        compiler_params=_PARAMS,
        scratch_types=[
            pltpu.VMEM((_EB,), jnp.int32),
            pltpu.VMEM((_EB,), jnp.int32),
            pltpu.VMEM((_EB,), jnp.float32),
            pltpu.VMEM((4, _STG), jnp.int32),
            pltpu.VMEM((4, _STG), jnp.int32),
            pltpu.VMEM((4, _STG), jnp.float32),
            pltpu.VMEM((_L,), jnp.int32),
        ],
    )


def _make_spmm():
    nsub = _BLK // _SUB
    return pl.kernel(
        _spmm_body,
        out_type=jax.ShapeDtypeStruct((_NPAD, _D), jnp.float32),
        mesh=_MESH,
        compiler_params=_PARAMS,
        scratch_types=[
            pltpu.VMEM((_L,), jnp.int32),
            [pltpu.VMEM((_BLK,), jnp.int32) for _ in range(2)],
            [pltpu.VMEM((_BLK,), jnp.float32) for _ in range(2)],
            [pltpu.VMEM((nsub, _SUB), jnp.int32) for _ in range(2)],
            [pltpu.VMEM((nsub, _SUB), jnp.int32) for _ in range(2)],
            [pltpu.VMEM((_BLK, _D), jnp.float32) for _ in range(2)]
            + [pltpu.VMEM_SHARED((_CH, _D), jnp.float32)],
            [pltpu.SemaphoreType.DMA for _ in range(2)],
            [pltpu.SemaphoreType.DMA for _ in range(2)],
            [pltpu.SemaphoreType.DMA for _ in range(2)],
        ],
    )


def _make_final():
    nb = _B // _NW
    return pl.kernel(
        _final_body,
        out_type=jax.ShapeDtypeStruct((_B,), jnp.float32),
        mesh=_MESH,
        compiler_params=_PARAMS,
        scratch_types=[
            pltpu.VMEM((nb,), jnp.int32),
            pltpu.VMEM((nb,), jnp.int32),
            pltpu.VMEM((nb, _D), jnp.float32),
            pltpu.VMEM((nb, _D), jnp.float32),
            pltpu.VMEM((nb, _D), jnp.float32),
            pltpu.VMEM((nb, _D), jnp.float32),
            pltpu.VMEM((nb, _D), jnp.float32),
            pltpu.VMEM((nb, _D), jnp.float32),
            pltpu.VMEM((nb, _D), jnp.float32),
            pltpu.VMEM((nb, _D), jnp.float32),
            pltpu.VMEM((nb,), jnp.float32),
        ],
    )


def kernel(user_ids, item_ids, user_emb, item_emb, adj_row, adj_col, adj_vals):
    e0 = jnp.concatenate([user_emb, item_emb], axis=0)
    e0 = jnp.pad(e0, ((0, _NPAD - _N), (0, 0)))
    pad = _NNZ32 - _NNZ
    rows = jnp.concatenate([adj_row, jnp.arange(pad, dtype=jnp.int32) % _N])
    cols = jnp.pad(adj_col, (0, pad))
    vals = jnp.pad(adj_vals, (0, pad))
    zrows = jnp.zeros((_ZROWS, _D), jnp.float32)

    prow, pcol, pval, meta = _make_part()(rows, cols, vals)

    spmm = _make_spmm()
    e1 = spmm(e0, prow, pcol, pval, meta, zrows)
    e2 = spmm(e1, prow, pcol, pval, meta, zrows)
    e3 = spmm(e2, prow, pcol, pval, meta, zrows)

    return _make_final()(e0, e1, e2, e3, user_ids, item_ids)
